# initial kernel scaffold (unmeasured)
import jax
import jax.numpy as jnp
from jax import lax
from jax.experimental import pallas as pl
from jax.experimental.pallas import tpu as pltpu

N_DEV = 4


def kernel(x, w_mat):
    m_full, k_per = x.shape
    _, n = w_mat.shape
    m_per = m_full // N_DEV
    TN = 1024
    NT = n // TN

    def body(x_hbm, w_ref, y_ref, scale_ref,
             comm_ref, x_vmem, amax_send, amax_recv,
             x_sem, send_sems, recv_sems, amax_send_sems, amax_recv_sems,
             credit_sem, amax_smem):
        s = pl.program_id(0)
        t = pl.program_id(1)
        my = lax.axis_index("i")
        left = (my + N_DEV - 1) % N_DEV
        right = (my + 1) % N_DEV

        def ring_rdma(step):
            return pltpu.make_async_remote_copy(
                src_ref=comm_ref.at[(step + 1) % 2],
                dst_ref=comm_ref.at[step % 2],
                send_sem=send_sems.at[step],
                recv_sem=recv_sems.at[step],
                device_id=(right,),
                device_id_type=pl.DeviceIdType.MESH,
            )

        @pl.when(jnp.logical_and(s == 0, t == 0))
        def _entry():
            bar = pltpu.get_barrier_semaphore()
            for nbr in (left, right):
                pl.semaphore_signal(bar, inc=1, device_id=(nbr,),
                                    device_id_type=pl.DeviceIdType.MESH)
            pl.semaphore_wait(bar, 2)
            amax_smem[0] = 0.0

        @pl.when(t == 0)
        def _fetch_x():
            j = (my + 2 * N_DEV - 1 - s) % N_DEV
            cp = pltpu.make_async_copy(
                x_hbm.at[pl.ds(j * m_per, m_per), :], x_vmem, x_sem)
            cp.start()
            cp.wait()

        @pl.when(jnp.logical_and(s >= 1, t == 0))
        def _sync_prev():
            prev = s - 1
            ring_rdma(prev).wait_recv()
            ring_rdma(prev).wait_send()

            @pl.when(prev <= 1)
            def _credit():
                pl.semaphore_signal(credit_sem, inc=1, device_id=(left,),
                                    device_id_type=pl.DeviceIdType.MESH)

        p = jnp.dot(x_vmem[:, :], w_ref[:, :],
                    preferred_element_type=jnp.float32)

        rw = (s + 1) % 2

        @pl.when(s == 0)
        def _store0():
            comm_ref[rw, t] = p.astype(jnp.bfloat16)

        @pl.when(jnp.logical_and(s >= 1, s <= 2))
        def _accum():
            acc = p + comm_ref[rw, t].astype(jnp.float32)
            comm_ref[rw, t] = acc.astype(jnp.bfloat16)

        @pl.when(s == 3)
        def _final():
            yv = p + comm_ref[rw, t].astype(jnp.float32)
            y_ref[:, :] = yv
            amax_smem[0] = jnp.maximum(amax_smem[0], jnp.max(jnp.abs(yv)))

        @pl.when(jnp.logical_and(t == NT - 1, s < 3))
        def _send():
            @pl.when(s >= 1)
            def _():
                pl.semaphore_wait(credit_sem, 1)
            ring_rdma(s).start()

        @pl.when(jnp.logical_and(s == 3, t == NT - 1))
        def _finish():
            ring_rdma(2).wait_send()
            amax_send[:, :] = jnp.full((8, 128), amax_smem[0], jnp.float32)

            def amax_rdma(o):
                peer = (my + o) % N_DEV
                return pltpu.make_async_remote_copy(
                    src_ref=amax_send,
                    dst_ref=amax_recv.at[o - 1],
                    send_sem=amax_send_sems.at[o - 1],
                    recv_sem=amax_recv_sems.at[o - 1],
                    device_id=(peer,),
                    device_id_type=pl.DeviceIdType.MESH,
                )

            for o in (1, 2, 3):
                amax_rdma(o).start()
            g = amax_smem[0]
            for o in (1, 2, 3):
                d = amax_rdma(o)
                d.wait_send()
                d.wait_recv()
                g = jnp.maximum(g, jnp.max(amax_recv[o - 1]))
            scale_ref[:, :] = jnp.full((1, 1), g / 127.0, jnp.float32)

    y_raw, scale = pl.pallas_call(
        body,
        grid=(N_DEV, NT),
        in_specs=[
            pl.BlockSpec(memory_space=pltpu.ANY),
            pl.BlockSpec((k_per, TN), lambda s, t: (0, t)),
        ],
        out_specs=[
            pl.BlockSpec((m_per, TN), lambda s, t: (0, t)),
            pl.BlockSpec((1, 1), lambda s, t: (0, 0)),
        ],
        out_shape=[
            jax.ShapeDtypeStruct((m_per, n), jnp.float32),
            jax.ShapeDtypeStruct((1, 1), jnp.float32),
        ],
        scratch_shapes=[
            pltpu.VMEM((2, NT, m_per, TN), jnp.bfloat16),
            pltpu.VMEM((m_per, k_per), jnp.float32),
            pltpu.VMEM((8, 128), jnp.float32),
            pltpu.VMEM((3, 8, 128), jnp.float32),
            pltpu.SemaphoreType.DMA,
            pltpu.SemaphoreType.DMA((3,)),
            pltpu.SemaphoreType.DMA((3,)),
            pltpu.SemaphoreType.DMA((3,)),
            pltpu.SemaphoreType.DMA((3,)),
            pltpu.SemaphoreType.REGULAR,
            pltpu.SMEM((1,), jnp.float32),
        ],
        compiler_params=pltpu.CompilerParams(
            collective_id=0,
            dimension_semantics=("arbitrary", "arbitrary"),
        ),
    )(x, w_mat)

    def epi_body(y_in, scale_in, out_ref):
        sc = scale_in[:, :]
        q = jnp.clip(jnp.round(y_in[:, :] / sc), -127.0, 127.0)
        out_ref[:, :] = q * sc

    return pl.pallas_call(
        epi_body,
        grid=(NT,),
        in_specs=[
            pl.BlockSpec((m_per, TN), lambda t: (0, t)),
            pl.BlockSpec((1, 1), lambda t: (0, 0)),
        ],
        out_specs=pl.BlockSpec((m_per, TN), lambda t: (0, t)),
        out_shape=jax.ShapeDtypeStruct((m_per, n), jnp.float32),
    )(y_raw, scale)


# baseline (device time: 645537 ns/iter reference)
import os

import jax
import jax.numpy as jnp
from jax import lax
from jax.experimental import pallas as pl
from jax.experimental.pallas import tpu as pltpu

N_DEV = 4
LEVEL = int(os.environ.get("KDBG", "3"))
NO_CREDIT = os.environ.get("KDBG_NOCREDIT", "0") == "1"


def kernel(x, w_mat):
    m_full, k_per = x.shape
    _, n = w_mat.shape
    m_per = m_full // N_DEV
    TN = 1024
    NT = n // TN

    def body(x_hbm, w_ref, y_ref, scale_ref,
             comm_ref, x_vmem, amax_send, amax_recv, credit_buf,
             x_sem, ring_send_sem, ring_recv_sems, amax_send_sem,
             amax_recv_sem, credit_send_sem, credit_recv_sem, amax_smem):
        s = pl.program_id(0)
        t = pl.program_id(1)
        my = lax.axis_index("i")
        left = (my + N_DEV - 1) % N_DEV
        right = (my + 1) % N_DEV

        def tile_rdma(st: int, tt: int):
            return pltpu.make_async_remote_copy(
                src_ref=comm_ref.at[(st + 1) % 2, tt],
                dst_ref=comm_ref.at[st % 2, tt],
                send_sem=ring_send_sem,
                recv_sem=ring_recv_sems.at[st],
                device_id=(right,),
                device_id_type=pl.DeviceIdType.MESH,
            )

        def credit_rdma(k: int):
            return pltpu.make_async_remote_copy(
                src_ref=credit_buf.at[0],
                dst_ref=credit_buf.at[1 + k],
                send_sem=credit_send_sem,
                recv_sem=credit_recv_sem,
                device_id=(left,),
                device_id_type=pl.DeviceIdType.MESH,
            )

        @pl.when(jnp.logical_and(s == 0, t == 0))
        def _entry():
            if LEVEL >= 1:
                bar = pltpu.get_barrier_semaphore()
                for nbr in (left, right):
                    pl.semaphore_signal(bar, inc=1, device_id=(nbr,),
                                        device_id_type=pl.DeviceIdType.MESH)
                pl.semaphore_wait(bar, 2)
            amax_smem[0] = 0.0

        @pl.when(t == 0)
        def _fetch_x():
            j = (my + 2 * N_DEV - 1 - s) % N_DEV
            cp = pltpu.make_async_copy(
                x_hbm.at[pl.ds(j * m_per, m_per), :], x_vmem, x_sem)
            cp.start()
            cp.wait()

        if LEVEL >= 2:
            for prev in range(3):
                @pl.when(jnp.logical_and(s == prev + 1, t == 0))
                def _retire_prev(prev=prev):
                    for tt in range(NT):
                        tile_rdma(prev, tt).wait_send()
                    if prev <= 1 and not NO_CREDIT:
                        if prev == 1:
                            credit_rdma(0).wait_send()
                        credit_rdma(prev).start()
                    for tt in range(NT):
                        tile_rdma(prev, tt).wait_recv()

        p = jnp.dot(x_vmem[:, :], w_ref[:, :],
                    preferred_element_type=jnp.float32)

        for st in range(4):
            rw = (st + 1) % 2

            @pl.when(s == st)
            def _compute(st=st, rw=rw):
                if st == 0:
                    comm_ref[rw, t] = p.astype(jnp.bfloat16)
                elif st <= 2:
                    if LEVEL >= 2:
                        acc = p + comm_ref[rw, t].astype(jnp.float32)
                    else:
                        acc = p
                    comm_ref[rw, t] = acc.astype(jnp.bfloat16)
                else:
                    if LEVEL >= 2:
                        yv = p + comm_ref[rw, t].astype(jnp.float32)
                    else:
                        yv = p
                    y_ref[:, :] = yv
                    amax_smem[0] = jnp.maximum(
                        amax_smem[0], jnp.max(jnp.abs(yv)))

        if LEVEL >= 2:
            for st in range(3):
                for tt in range(NT):
                    @pl.when(jnp.logical_and(s == st, t == tt))
                    def _send(st=st, tt=tt):
                        if st >= 1 and tt == 0 and not NO_CREDIT:
                            credit_rdma(st - 1).wait_recv()
                        tile_rdma(st, tt).start()

        @pl.when(jnp.logical_and(s == 3, t == NT - 1))
        def _finish():
            if LEVEL >= 2 and not NO_CREDIT:
                credit_rdma(1).wait_send()
            g = amax_smem[0]
            if LEVEL >= 3:
                amax_send[:, :] = jnp.full((8, 128), g, jnp.float32)

                def amax_rdma(o):
                    peer = (my + o) % N_DEV
                    return pltpu.make_async_remote_copy(
                        src_ref=amax_send,
                        dst_ref=amax_recv.at[o - 1],
                        send_sem=amax_send_sem,
                        recv_sem=amax_recv_sem,
                        device_id=(peer,),
                        device_id_type=pl.DeviceIdType.MESH,
                    )

                for o in (1, 2, 3):
                    amax_rdma(o).start()
                for o in (1, 2, 3):
                    d = amax_rdma(o)
                    d.wait_send()
                    d.wait_recv()
                for o in (1, 2, 3):
                    g = jnp.maximum(g, jnp.max(amax_recv[o - 1]))
            scale_ref[:, :] = jnp.full((1, 1), g / 127.0, jnp.float32)

    y_raw, scale = pl.pallas_call(
        body,
        grid=(N_DEV, NT),
        in_specs=[
            pl.BlockSpec(memory_space=pl.ANY),
            pl.BlockSpec((k_per, TN), lambda s, t: (0, t)),
        ],
        out_specs=[
            pl.BlockSpec((m_per, TN), lambda s, t: (0, t)),
            pl.BlockSpec((1, 1), lambda s, t: (0, 0)),
        ],
        out_shape=[
            jax.ShapeDtypeStruct((m_per, n), jnp.float32),
            jax.ShapeDtypeStruct((1, 1), jnp.float32),
        ],
        scratch_shapes=[
            pltpu.VMEM((2, NT, m_per, TN), jnp.bfloat16),
            pltpu.VMEM((m_per, k_per), jnp.bfloat16),
            pltpu.VMEM((8, 128), jnp.float32),
            pltpu.VMEM((3, 8, 128), jnp.float32),
            pltpu.VMEM((3, 8, 128), jnp.float32),
            pltpu.SemaphoreType.DMA,
            pltpu.SemaphoreType.DMA,
            pltpu.SemaphoreType.DMA((3,)),
            pltpu.SemaphoreType.DMA,
            pltpu.SemaphoreType.DMA,
            pltpu.SemaphoreType.DMA,
            pltpu.SemaphoreType.DMA,
            pltpu.SMEM((1,), jnp.float32),
        ],
        compiler_params=pltpu.CompilerParams(
            collective_id=0 if LEVEL >= 1 else None,
            dimension_semantics=("arbitrary", "arbitrary"),
            vmem_limit_bytes=64 * 1024 * 1024,
        ),
    )(x.astype(jnp.bfloat16), w_mat.astype(jnp.bfloat16))

    def epi_body(y_in, scale_in, out_ref):
        sc = scale_in[:, :]
        q = jnp.clip(jnp.round(y_in[:, :] / sc), -127.0, 127.0)
        out_ref[:, :] = q * sc

    return pl.pallas_call(
        epi_body,
        grid=(NT,),
        in_specs=[
            pl.BlockSpec((m_per, TN), lambda t: (0, t)),
            pl.BlockSpec((1, 1), lambda t: (0, 0)),
        ],
        out_specs=pl.BlockSpec((m_per, TN), lambda t: (0, t)),
        out_shape=jax.ShapeDtypeStruct((m_per, n), jnp.float32),
    )(y_raw, scale)


# device time: 414477 ns/iter; 1.5575x vs baseline; 1.5575x over previous
import os

import jax
import jax.numpy as jnp
from jax import lax
from jax.experimental import pallas as pl
from jax.experimental.pallas import tpu as pltpu

N_DEV = 4
LEVEL = int(os.environ.get("KDBG", "3"))
NO_CREDIT = os.environ.get("KDBG_NOCREDIT", "0") == "1"


def kernel(x, w_mat):
    m_full, k_per = x.shape
    _, n = w_mat.shape
    m_per = m_full // N_DEV
    TN = 1024
    NT = n // TN

    def body(x_hbm, w_ref, y_ref, scale_ref,
             comm_ref, x_vmem, amax_send, amax_recv, credit_buf,
             x_sem, ring_send_sem, ring_recv_sems, amax_send_sem,
             amax_recv_sem, credit_send_sem, credit_recv_sems, amax_smem):
        s = pl.program_id(0)
        t = pl.program_id(1)
        my = lax.axis_index("i")
        left = (my + N_DEV - 1) % N_DEV
        right = (my + 1) % N_DEV

        def tile_rdma(st: int, tt: int):
            return pltpu.make_async_remote_copy(
                src_ref=comm_ref.at[(st + 1) % 2, tt],
                dst_ref=comm_ref.at[st % 2, tt],
                send_sem=ring_send_sem,
                recv_sem=ring_recv_sems.at[st],
                device_id=(right,) if tt < NT // 2 else (left,),
                device_id_type=pl.DeviceIdType.MESH,
            )

        def credit_rdma(k: int, d: int):
            return pltpu.make_async_remote_copy(
                src_ref=credit_buf.at[0],
                dst_ref=credit_buf.at[1 + 2 * k + d],
                send_sem=credit_send_sem,
                recv_sem=credit_recv_sems.at[d],
                device_id=(left,) if d == 0 else (right,),
                device_id_type=pl.DeviceIdType.MESH,
            )

        @pl.when(jnp.logical_and(s == 0, t == 0))
        def _entry():
            if LEVEL >= 1:
                bar = pltpu.get_barrier_semaphore()
                for nbr in (left, right):
                    pl.semaphore_signal(bar, inc=1, device_id=(nbr,),
                                        device_id_type=pl.DeviceIdType.MESH)
                pl.semaphore_wait(bar, 2)
            amax_smem[0] = 0.0

        @pl.when(t == 0)
        def _fetch_x():
            j0 = (my + 2 * N_DEV - 1 - s) % N_DEV
            j1 = (my + 1 + s) % N_DEV
            cp0 = pltpu.make_async_copy(
                x_hbm.at[pl.ds(j0 * m_per, m_per), :], x_vmem.at[0], x_sem)
            cp1 = pltpu.make_async_copy(
                x_hbm.at[pl.ds(j1 * m_per, m_per), :], x_vmem.at[1], x_sem)
            cp0.start()
            cp1.start()
            cp0.wait()
            cp1.wait()

        if LEVEL >= 2:
            for prev in range(3):
                @pl.when(jnp.logical_and(s == prev + 1, t == 0))
                def _retire_prev(prev=prev):
                    for tt in range(NT):
                        tile_rdma(prev, tt).wait_send()
                    if prev <= 1 and not NO_CREDIT:
                        if prev == 1:
                            credit_rdma(0, 0).wait_send()
                            credit_rdma(0, 1).wait_send()
                        credit_rdma(prev, 0).start()
                        credit_rdma(prev, 1).start()
                    for tt in range(NT):
                        tile_rdma(prev, tt).wait_recv()

        xa = x_vmem[t // (NT // 2)]
        p = jnp.dot(xa, w_ref[:, :], preferred_element_type=jnp.float32)

        for st in range(4):
            rw = (st + 1) % 2

            @pl.when(s == st)
            def _compute(st=st, rw=rw):
                if st == 0:
                    comm_ref[rw, t] = p.astype(jnp.bfloat16)
                elif st <= 2:
                    if LEVEL >= 2:
                        acc = p + comm_ref[rw, t].astype(jnp.float32)
                    else:
                        acc = p
                    comm_ref[rw, t] = acc.astype(jnp.bfloat16)
                else:
                    if LEVEL >= 2:
                        yv = p + comm_ref[rw, t].astype(jnp.float32)
                    else:
                        yv = p
                    y_ref[:, :] = yv
                    amax_smem[0] = jnp.maximum(
                        amax_smem[0], jnp.max(jnp.abs(yv)))

        if LEVEL >= 2:
            for st in range(3):
                for tt in range(NT):
                    @pl.when(jnp.logical_and(s == st, t == tt))
                    def _send(st=st, tt=tt):
                        if st >= 1 and tt in (0, NT // 2) and not NO_CREDIT:
                            credit_rdma(st - 1, 0 if tt == 0 else 1).wait_recv()
                        tile_rdma(st, tt).start()

        @pl.when(jnp.logical_and(s == 3, t == NT - 1))
        def _finish():
            if LEVEL >= 2 and not NO_CREDIT:
                credit_rdma(1, 0).wait_send()
                credit_rdma(1, 1).wait_send()
            g = amax_smem[0]
            if LEVEL >= 3:
                amax_send[:, :] = jnp.full((8, 128), g, jnp.float32)

                def amax_rdma(o):
                    peer = (my + o) % N_DEV
                    return pltpu.make_async_remote_copy(
                        src_ref=amax_send,
                        dst_ref=amax_recv.at[o - 1],
                        send_sem=amax_send_sem,
                        recv_sem=amax_recv_sem,
                        device_id=(peer,),
                        device_id_type=pl.DeviceIdType.MESH,
                    )

                for o in (1, 2, 3):
                    amax_rdma(o).start()
                for o in (1, 2, 3):
                    d = amax_rdma(o)
                    d.wait_send()
                    d.wait_recv()
                for o in (1, 2, 3):
                    g = jnp.maximum(g, jnp.max(amax_recv[o - 1]))
            scale_ref[:, :] = jnp.full((1, 1), g / 127.0, jnp.float32)

    y_raw, scale = pl.pallas_call(
        body,
        grid=(N_DEV, NT),
        in_specs=[
            pl.BlockSpec(memory_space=pl.ANY),
            pl.BlockSpec((k_per, TN), lambda s, t: (0, t)),
        ],
        out_specs=[
            pl.BlockSpec((m_per, TN), lambda s, t: (0, t)),
            pl.BlockSpec((1, 1), lambda s, t: (0, 0)),
        ],
        out_shape=[
            jax.ShapeDtypeStruct((m_per, n), jnp.float32),
            jax.ShapeDtypeStruct((1, 1), jnp.float32),
        ],
        scratch_shapes=[
            pltpu.VMEM((2, NT, m_per, TN), jnp.bfloat16),
            pltpu.VMEM((2, m_per, k_per), jnp.bfloat16),
            pltpu.VMEM((8, 128), jnp.float32),
            pltpu.VMEM((3, 8, 128), jnp.float32),
            pltpu.VMEM((5, 8, 128), jnp.float32),
            pltpu.SemaphoreType.DMA,
            pltpu.SemaphoreType.DMA,
            pltpu.SemaphoreType.DMA((3,)),
            pltpu.SemaphoreType.DMA,
            pltpu.SemaphoreType.DMA,
            pltpu.SemaphoreType.DMA,
            pltpu.SemaphoreType.DMA((2,)),
            pltpu.SMEM((1,), jnp.float32),
        ],
        compiler_params=pltpu.CompilerParams(
            collective_id=0 if LEVEL >= 1 else None,
            dimension_semantics=("arbitrary", "arbitrary"),
            vmem_limit_bytes=64 * 1024 * 1024,
        ),
    )(x.astype(jnp.bfloat16), w_mat.astype(jnp.bfloat16))

    def epi_body(y_in, scale_in, out_ref):
        sc = scale_in[:, :]
        q = jnp.clip(jnp.round(y_in[:, :] / sc), -127.0, 127.0)
        out_ref[:, :] = q * sc

    return pl.pallas_call(
        epi_body,
        grid=(NT,),
        in_specs=[
            pl.BlockSpec((m_per, TN), lambda t: (0, t)),
            pl.BlockSpec((1, 1), lambda t: (0, 0)),
        ],
        out_specs=pl.BlockSpec((m_per, TN), lambda t: (0, t)),
        out_shape=jax.ShapeDtypeStruct((m_per, n), jnp.float32),
    )(y_raw, scale)


# device time: 412101 ns/iter; 1.5665x vs baseline; 1.0058x over previous
import os

import jax
import jax.numpy as jnp
from jax import lax
from jax.experimental import pallas as pl
from jax.experimental.pallas import tpu as pltpu

N_DEV = 4
LEVEL = int(os.environ.get("KDBG", "3"))
NO_CREDIT = os.environ.get("KDBG_NOCREDIT", "0") == "1"


def kernel(x, w_mat):
    m_full, k_per = x.shape
    _, n = w_mat.shape
    m_per = m_full // N_DEV
    TN = 1024
    NT = n // TN

    def body(x_hbm, w_ref, y_ref, scale_ref,
             comm_ref, x_vmem, amax_send, amax_recv, credit_buf,
             x_sem, ring_send_sem, ring_recv_sems, amax_send_sem,
             amax_recv_sem, credit_send_sem, credit_recv_sems, amax_smem):
        s = pl.program_id(0)
        t = pl.program_id(1)
        my = lax.axis_index("i")
        left = (my + N_DEV - 1) % N_DEV
        right = (my + 1) % N_DEV

        def tile_rdma(st: int, tt: int):
            return pltpu.make_async_remote_copy(
                src_ref=comm_ref.at[(st + 1) % 2, tt],
                dst_ref=comm_ref.at[st % 2, tt],
                send_sem=ring_send_sem,
                recv_sem=ring_recv_sems.at[st, 0 if tt < NT // 2 else 1],
                device_id=(right,) if tt < NT // 2 else (left,),
                device_id_type=pl.DeviceIdType.MESH,
            )

        def credit_rdma(k: int, d: int):
            return pltpu.make_async_remote_copy(
                src_ref=credit_buf.at[0],
                dst_ref=credit_buf.at[1 + 2 * k + d],
                send_sem=credit_send_sem,
                recv_sem=credit_recv_sems.at[d],
                device_id=(left,) if d == 0 else (right,),
                device_id_type=pl.DeviceIdType.MESH,
            )

        @pl.when(jnp.logical_and(s == 0, t == 0))
        def _entry():
            if LEVEL >= 1:
                bar = pltpu.get_barrier_semaphore()
                for nbr in (left, right):
                    pl.semaphore_signal(bar, inc=1, device_id=(nbr,),
                                        device_id_type=pl.DeviceIdType.MESH)
                pl.semaphore_wait(bar, 2)
            amax_smem[0] = 0.0

        @pl.when(t == 0)
        def _fetch_x():
            j0 = (my + 2 * N_DEV - 1 - s) % N_DEV
            j1 = (my + 1 + s) % N_DEV
            cp0 = pltpu.make_async_copy(
                x_hbm.at[pl.ds(j0 * m_per, m_per), :], x_vmem.at[0], x_sem)
            cp1 = pltpu.make_async_copy(
                x_hbm.at[pl.ds(j1 * m_per, m_per), :], x_vmem.at[1], x_sem)
            cp0.start()
            cp1.start()
            cp0.wait()
            cp1.wait()

        if LEVEL >= 2:
            for prev in range(3):
                @pl.when(jnp.logical_and(s == prev + 1, t == 0))
                def _retire_prev(prev=prev):
                    for tt in range(NT):
                        tile_rdma(prev, tt).wait_send()
                    if prev <= 1 and not NO_CREDIT:
                        if prev == 1:
                            credit_rdma(0, 0).wait_send()
                            credit_rdma(0, 1).wait_send()
                        credit_rdma(prev, 0).start()
                        credit_rdma(prev, 1).start()

            for prev in range(3):
                for tt in range(NT):
                    @pl.when(jnp.logical_and(s == prev + 1, t == tt))
                    def _wait_tile(prev=prev, tt=tt):
                        tile_rdma(prev, tt).wait_recv()

        xa = x_vmem[t // (NT // 2)]
        p = jnp.dot(xa, w_ref[:, :], preferred_element_type=jnp.float32)

        for st in range(4):
            rw = (st + 1) % 2

            @pl.when(s == st)
            def _compute(st=st, rw=rw):
                if st == 0:
                    comm_ref[rw, t] = p.astype(jnp.bfloat16)
                elif st <= 2:
                    if LEVEL >= 2:
                        acc = p + comm_ref[rw, t].astype(jnp.float32)
                    else:
                        acc = p
                    comm_ref[rw, t] = acc.astype(jnp.bfloat16)
                else:
                    if LEVEL >= 2:
                        yv = p + comm_ref[rw, t].astype(jnp.float32)
                    else:
                        yv = p
                    y_ref[:, :] = yv
                    amax_smem[0] = jnp.maximum(
                        amax_smem[0], jnp.max(jnp.abs(yv)))

        if LEVEL >= 2:
            for st in range(3):
                for tt in range(NT):
                    @pl.when(jnp.logical_and(s == st, t == tt))
                    def _send(st=st, tt=tt):
                        if st >= 1 and tt in (0, NT // 2) and not NO_CREDIT:
                            credit_rdma(st - 1, 0 if tt == 0 else 1).wait_recv()
                        tile_rdma(st, tt).start()

        @pl.when(jnp.logical_and(s == 3, t == NT - 1))
        def _finish():
            if LEVEL >= 2 and not NO_CREDIT:
                credit_rdma(1, 0).wait_send()
                credit_rdma(1, 1).wait_send()
            g = amax_smem[0]
            if LEVEL >= 3:
                amax_send[:, :] = jnp.full((8, 128), g, jnp.float32)

                def amax_rdma(o):
                    peer = (my + o) % N_DEV
                    return pltpu.make_async_remote_copy(
                        src_ref=amax_send,
                        dst_ref=amax_recv.at[o - 1],
                        send_sem=amax_send_sem,
                        recv_sem=amax_recv_sem,
                        device_id=(peer,),
                        device_id_type=pl.DeviceIdType.MESH,
                    )

                for o in (1, 2, 3):
                    amax_rdma(o).start()
                for o in (1, 2, 3):
                    d = amax_rdma(o)
                    d.wait_send()
                    d.wait_recv()
                for o in (1, 2, 3):
                    g = jnp.maximum(g, jnp.max(amax_recv[o - 1]))
            scale_ref[:, :] = jnp.full((1, 1), g / 127.0, jnp.float32)

    y_raw, scale = pl.pallas_call(
        body,
        grid=(N_DEV, NT),
        in_specs=[
            pl.BlockSpec(memory_space=pl.ANY),
            pl.BlockSpec((k_per, TN), lambda s, t: (0, t)),
        ],
        out_specs=[
            pl.BlockSpec((m_per, TN), lambda s, t: (0, t)),
            pl.BlockSpec((1, 1), lambda s, t: (0, 0)),
        ],
        out_shape=[
            jax.ShapeDtypeStruct((m_per, n), jnp.float32),
            jax.ShapeDtypeStruct((1, 1), jnp.float32),
        ],
        scratch_shapes=[
            pltpu.VMEM((2, NT, m_per, TN), jnp.bfloat16),
            pltpu.VMEM((2, m_per, k_per), jnp.bfloat16),
            pltpu.VMEM((8, 128), jnp.float32),
            pltpu.VMEM((3, 8, 128), jnp.float32),
            pltpu.VMEM((5, 8, 128), jnp.float32),
            pltpu.SemaphoreType.DMA,
            pltpu.SemaphoreType.DMA,
            pltpu.SemaphoreType.DMA((3, 2)),
            pltpu.SemaphoreType.DMA,
            pltpu.SemaphoreType.DMA,
            pltpu.SemaphoreType.DMA,
            pltpu.SemaphoreType.DMA((2,)),
            pltpu.SMEM((1,), jnp.float32),
        ],
        compiler_params=pltpu.CompilerParams(
            collective_id=0 if LEVEL >= 1 else None,
            dimension_semantics=("arbitrary", "arbitrary"),
            vmem_limit_bytes=64 * 1024 * 1024,
        ),
    )(x.astype(jnp.bfloat16), w_mat.astype(jnp.bfloat16))

    def epi_body(y_in, scale_in, out_ref):
        sc = scale_in[:, :]
        q = jnp.clip(jnp.round(y_in[:, :] / sc), -127.0, 127.0)
        out_ref[:, :] = q * sc

    return pl.pallas_call(
        epi_body,
        grid=(NT,),
        in_specs=[
            pl.BlockSpec((m_per, TN), lambda t: (0, t)),
            pl.BlockSpec((1, 1), lambda t: (0, 0)),
        ],
        out_specs=pl.BlockSpec((m_per, TN), lambda t: (0, t)),
        out_shape=jax.ShapeDtypeStruct((m_per, n), jnp.float32),
    )(y_raw, scale)
